# baseline (speedup 1.0000x reference)
import jax
import jax.numpy as jnp
from jax import lax
from jax.experimental import pallas as pl
from jax.experimental.pallas import tpu as pltpu

B = 64
EDD = 2048
EDS = 1024
BLK = 128
NBLK = EDS // BLK
BN_EPS = 1e-5
COS_EPS = 1e-8

_DN_T = (((1,), (1,)), ((), ()))
_DN = (((1,), (0,)), ((), ()))


def _fused_kernel(x_ref, y_ref, w_ref, gx_ref, bx_ref, gy_ref, by_ref,
                  out_ref, acc_dot, acc_nx, acc_ny):
    j = pl.program_id(0)
    w = w_ref[...]
    hx = lax.dot_general(x_ref[...], w, _DN_T,
                         preferred_element_type=jnp.float32)
    hy = lax.dot_general(y_ref[...], w, _DN_T,
                         preferred_element_type=jnp.float32)

    ones_row = jnp.ones((1, B), dtype=jnp.float32)

    def bn_tanh(hh, g, bb):
        s1 = lax.dot_general(ones_row, hh, _DN,
                             preferred_element_type=jnp.float32)
        s2 = lax.dot_general(ones_row, hh * hh, _DN,
                             preferred_element_type=jnp.float32)
        mu = s1 * (1.0 / B)
        var = s2 * (1.0 / B) - mu * mu
        scale = lax.rsqrt(var + BN_EPS) * g
        shift = bb - mu * scale
        return jnp.tanh(hh * scale + shift)

    hx = bn_tanh(hx, gx_ref[...], bx_ref[...])
    hy = bn_tanh(hy, gy_ref[...], by_ref[...])

    lane = lax.broadcasted_iota(jnp.int32, (B, BLK), 1)
    at_block_start = (lane % 4) == 0
    neg_inf = jnp.full((B, BLK), -jnp.inf, dtype=jnp.float32)

    def block_mask(hh):
        a = jnp.maximum(hh, pltpu.roll(hh, BLK - 1, 1))
        bm = jnp.maximum(a, pltpu.roll(a, BLK - 2, 1))
        c = jnp.where(at_block_start, bm, neg_inf)
        c = jnp.maximum(c, pltpu.roll(c, 1, 1))
        bmax = jnp.maximum(c, pltpu.roll(c, 2, 1))
        return jnp.where(hh == bmax, hh, 0.0)

    mx = block_mask(hx)
    my = block_mask(hy)

    ones_col = jnp.ones((BLK, 1), dtype=jnp.float32)
    p_dot = lax.dot_general(mx * my, ones_col, _DN,
                            preferred_element_type=jnp.float32)
    p_nx = lax.dot_general(mx * mx, ones_col, _DN,
                           preferred_element_type=jnp.float32)
    p_ny = lax.dot_general(my * my, ones_col, _DN,
                           preferred_element_type=jnp.float32)

    @pl.when(j == 0)
    def _():
        acc_dot[...] = p_dot
        acc_nx[...] = p_nx
        acc_ny[...] = p_ny

    @pl.when(j != 0)
    def _():
        acc_dot[...] += p_dot
        acc_nx[...] += p_nx
        acc_ny[...] += p_ny

    @pl.when(j == NBLK - 1)
    def _():
        nx = jnp.maximum(jnp.sqrt(acc_nx[...]), COS_EPS)
        ny = jnp.maximum(jnp.sqrt(acc_ny[...]), COS_EPS)
        out_ref[...] = acc_dot[...] / (nx * ny)


def kernel(x, y, W, b, gamma_x, beta_x, gamma_y, beta_y):
    row = lambda v: v.reshape(1, EDS)
    out = pl.pallas_call(
        _fused_kernel,
        grid=(NBLK,),
        in_specs=[
            pl.BlockSpec((B, EDD), lambda j: (0, 0)),
            pl.BlockSpec((B, EDD), lambda j: (0, 0)),
            pl.BlockSpec((BLK, EDD), lambda j: (j, 0)),
            pl.BlockSpec((1, BLK), lambda j: (0, j)),
            pl.BlockSpec((1, BLK), lambda j: (0, j)),
            pl.BlockSpec((1, BLK), lambda j: (0, j)),
            pl.BlockSpec((1, BLK), lambda j: (0, j)),
        ],
        out_specs=pl.BlockSpec((B, 1), lambda j: (0, 0)),
        out_shape=jax.ShapeDtypeStruct((B, 1), jnp.float32),
        scratch_shapes=[pltpu.VMEM((B, 1), jnp.float32) for _ in range(3)],
        compiler_params=pltpu.CompilerParams(
            dimension_semantics=("arbitrary",)),
    )(x, y, W, row(gamma_x), row(beta_x), row(gamma_y), row(beta_y))
    return out.reshape(B)


# speedup vs baseline: 1.3219x; 1.3219x over previous
import jax
import jax.numpy as jnp
from jax import lax
from jax.experimental import pallas as pl
from jax.experimental.pallas import tpu as pltpu

B = 64
EDD = 2048
EDS = 1024
BLK = 512
NBLK = EDS // BLK
BN_EPS = 1e-5
COS_EPS = 1e-8

_DN_T = (((1,), (1,)), ((), ()))
_DN = (((1,), (0,)), ((), ()))


def _fused_kernel(x_ref, y_ref, w_ref, gx_ref, bx_ref, gy_ref, by_ref,
                  out_ref, acc_dot, acc_nx, acc_ny):
    j = pl.program_id(0)
    w = w_ref[...]
    hx = lax.dot_general(x_ref[...], w, _DN_T,
                         preferred_element_type=jnp.float32)
    hy = lax.dot_general(y_ref[...], w, _DN_T,
                         preferred_element_type=jnp.float32)

    ones_row = jnp.ones((1, B), dtype=jnp.float32)

    def bn_tanh(hh, g, bb):
        s1 = lax.dot_general(ones_row, hh, _DN,
                             preferred_element_type=jnp.float32)
        s2 = lax.dot_general(ones_row, hh * hh, _DN,
                             preferred_element_type=jnp.float32)
        mu = s1 * (1.0 / B)
        var = s2 * (1.0 / B) - mu * mu
        scale = lax.rsqrt(var + BN_EPS) * g
        shift = bb - mu * scale
        return jnp.tanh(hh * scale + shift)

    hx = bn_tanh(hx, gx_ref[...], bx_ref[...])
    hy = bn_tanh(hy, gy_ref[...], by_ref[...])

    lane = lax.broadcasted_iota(jnp.int32, (B, BLK), 1)
    at_block_start = (lane % 4) == 0
    neg_inf = jnp.full((B, BLK), -jnp.inf, dtype=jnp.float32)

    def block_mask(hh):
        a = jnp.maximum(hh, pltpu.roll(hh, BLK - 1, 1))
        bm = jnp.maximum(a, pltpu.roll(a, BLK - 2, 1))
        c = jnp.where(at_block_start, bm, neg_inf)
        c = jnp.maximum(c, pltpu.roll(c, 1, 1))
        bmax = jnp.maximum(c, pltpu.roll(c, 2, 1))
        return jnp.where(hh == bmax, hh, 0.0)

    mx = block_mask(hx)
    my = block_mask(hy)

    ones_col = jnp.ones((BLK, 1), dtype=jnp.float32)
    p_dot = lax.dot_general(mx * my, ones_col, _DN,
                            preferred_element_type=jnp.float32)
    p_nx = lax.dot_general(mx * mx, ones_col, _DN,
                           preferred_element_type=jnp.float32)
    p_ny = lax.dot_general(my * my, ones_col, _DN,
                           preferred_element_type=jnp.float32)

    @pl.when(j == 0)
    def _():
        acc_dot[...] = p_dot
        acc_nx[...] = p_nx
        acc_ny[...] = p_ny

    @pl.when(j != 0)
    def _():
        acc_dot[...] += p_dot
        acc_nx[...] += p_nx
        acc_ny[...] += p_ny

    @pl.when(j == NBLK - 1)
    def _():
        nx = jnp.maximum(jnp.sqrt(acc_nx[...]), COS_EPS)
        ny = jnp.maximum(jnp.sqrt(acc_ny[...]), COS_EPS)
        out_ref[...] = acc_dot[...] / (nx * ny)


def kernel(x, y, W, b, gamma_x, beta_x, gamma_y, beta_y):
    row = lambda v: v.reshape(1, EDS)
    out = pl.pallas_call(
        _fused_kernel,
        grid=(NBLK,),
        in_specs=[
            pl.BlockSpec((B, EDD), lambda j: (0, 0)),
            pl.BlockSpec((B, EDD), lambda j: (0, 0)),
            pl.BlockSpec((BLK, EDD), lambda j: (j, 0)),
            pl.BlockSpec((1, BLK), lambda j: (0, j)),
            pl.BlockSpec((1, BLK), lambda j: (0, j)),
            pl.BlockSpec((1, BLK), lambda j: (0, j)),
            pl.BlockSpec((1, BLK), lambda j: (0, j)),
        ],
        out_specs=pl.BlockSpec((B, 1), lambda j: (0, 0)),
        out_shape=jax.ShapeDtypeStruct((B, 1), jnp.float32),
        scratch_shapes=[pltpu.VMEM((B, 1), jnp.float32) for _ in range(3)],
        compiler_params=pltpu.CompilerParams(
            dimension_semantics=("arbitrary",)),
    )(x, y, W, row(gamma_x), row(beta_x), row(gamma_y), row(beta_y))
    return out.reshape(B)


# speedup vs baseline: 1.4032x; 1.0615x over previous
import jax
import jax.numpy as jnp
from jax import lax
from jax.experimental import pallas as pl
from jax.experimental.pallas import tpu as pltpu

B = 64
EDD = 2048
EDS = 1024
BLK = 1024
NBLK = EDS // BLK
BN_EPS = 1e-5
COS_EPS = 1e-8

_DN_T = (((1,), (1,)), ((), ()))
_DN = (((1,), (0,)), ((), ()))


def _fused_kernel(x_ref, y_ref, w_ref, gx_ref, bx_ref, gy_ref, by_ref,
                  out_ref, acc_dot, acc_nx, acc_ny):
    j = pl.program_id(0)
    w = w_ref[...]
    hx = lax.dot_general(x_ref[...], w, _DN_T,
                         preferred_element_type=jnp.float32)
    hy = lax.dot_general(y_ref[...], w, _DN_T,
                         preferred_element_type=jnp.float32)

    ones_row = jnp.ones((1, B), dtype=jnp.float32)

    def bn_tanh(hh, g, bb):
        s1 = lax.dot_general(ones_row, hh, _DN,
                             preferred_element_type=jnp.float32)
        s2 = lax.dot_general(ones_row, hh * hh, _DN,
                             preferred_element_type=jnp.float32)
        mu = s1 * (1.0 / B)
        var = s2 * (1.0 / B) - mu * mu
        scale = lax.rsqrt(var + BN_EPS) * g
        shift = bb - mu * scale
        return jnp.tanh(hh * scale + shift)

    hx = bn_tanh(hx, gx_ref[...], bx_ref[...])
    hy = bn_tanh(hy, gy_ref[...], by_ref[...])

    lane = lax.broadcasted_iota(jnp.int32, (B, BLK), 1)
    at_block_start = (lane % 4) == 0
    neg_inf = jnp.full((B, BLK), -jnp.inf, dtype=jnp.float32)

    def block_mask(hh):
        a = jnp.maximum(hh, pltpu.roll(hh, BLK - 1, 1))
        bm = jnp.maximum(a, pltpu.roll(a, BLK - 2, 1))
        c = jnp.where(at_block_start, bm, neg_inf)
        c = jnp.maximum(c, pltpu.roll(c, 1, 1))
        bmax = jnp.maximum(c, pltpu.roll(c, 2, 1))
        return jnp.where(hh == bmax, hh, 0.0)

    mx = block_mask(hx)
    my = block_mask(hy)

    ones_col = jnp.ones((BLK, 1), dtype=jnp.float32)
    p_dot = lax.dot_general(mx * my, ones_col, _DN,
                            preferred_element_type=jnp.float32)
    p_nx = lax.dot_general(mx * mx, ones_col, _DN,
                           preferred_element_type=jnp.float32)
    p_ny = lax.dot_general(my * my, ones_col, _DN,
                           preferred_element_type=jnp.float32)

    @pl.when(j == 0)
    def _():
        acc_dot[...] = p_dot
        acc_nx[...] = p_nx
        acc_ny[...] = p_ny

    @pl.when(j != 0)
    def _():
        acc_dot[...] += p_dot
        acc_nx[...] += p_nx
        acc_ny[...] += p_ny

    @pl.when(j == NBLK - 1)
    def _():
        nx = jnp.maximum(jnp.sqrt(acc_nx[...]), COS_EPS)
        ny = jnp.maximum(jnp.sqrt(acc_ny[...]), COS_EPS)
        out_ref[...] = acc_dot[...] / (nx * ny)


def kernel(x, y, W, b, gamma_x, beta_x, gamma_y, beta_y):
    row = lambda v: v.reshape(1, EDS)
    out = pl.pallas_call(
        _fused_kernel,
        grid=(NBLK,),
        in_specs=[
            pl.BlockSpec((B, EDD), lambda j: (0, 0)),
            pl.BlockSpec((B, EDD), lambda j: (0, 0)),
            pl.BlockSpec((BLK, EDD), lambda j: (j, 0)),
            pl.BlockSpec((1, BLK), lambda j: (0, j)),
            pl.BlockSpec((1, BLK), lambda j: (0, j)),
            pl.BlockSpec((1, BLK), lambda j: (0, j)),
            pl.BlockSpec((1, BLK), lambda j: (0, j)),
        ],
        out_specs=pl.BlockSpec((B, 1), lambda j: (0, 0)),
        out_shape=jax.ShapeDtypeStruct((B, 1), jnp.float32),
        scratch_shapes=[pltpu.VMEM((B, 1), jnp.float32) for _ in range(3)],
        compiler_params=pltpu.CompilerParams(
            dimension_semantics=("arbitrary",)),
    )(x, y, W, row(gamma_x), row(beta_x), row(gamma_y), row(beta_y))
    return out.reshape(B)
